# Initial kernel scaffold; baseline (speedup 1.0000x reference)
import jax
import jax.numpy as jnp
from jax.experimental import pallas as pl


def kernel(node_features, edge_index, W1, b1, W2, b2, W3, b3):
    raise NotImplementedError("write your pallas kernel here")


# speedup vs baseline: 1.0000x; 1.0000x over previous
import jax
import jax.numpy as jnp


def _bdot(a, b):
    return jax.lax.dot(a.astype(jnp.bfloat16), b.astype(jnp.bfloat16),
                       precision=jax.lax.Precision.HIGHEST,
                       preferred_element_type=jnp.float32)


def kernel(node_features, edge_index, W1, b1, W2, b2, W3, b3):
    src = edge_index[0]
    dst = edge_index[1]
    agg1 = jnp.zeros_like(node_features).at[dst].add(
        jnp.take(node_features, src, axis=0))
    hidden_rep = _bdot(agg1, W1) + b1
    agg2 = jnp.zeros((node_features.shape[0], W2.shape[0]),
                     dtype=hidden_rep.dtype).at[dst].add(
        jnp.take(hidden_rep, src, axis=0))
    node_embeddings = _bdot(agg2, W2) + b2
    source_embeddings = jnp.take(node_embeddings, src, axis=0)
    target_embeddings = jnp.take(node_embeddings, dst, axis=0)
    link_features = jnp.concatenate(
        [source_embeddings, target_embeddings], axis=1)
    return jax.nn.sigmoid(_bdot(link_features, W3) + b3)


# speedup vs baseline: 2.9279x; 2.9278x over previous
import functools

import jax
import jax.numpy as jnp
from jax import lax
from jax.experimental import pallas as pl
from jax.experimental.pallas import tpu as pltpu
from jax.experimental.pallas import tpu_sc as plsc

N_NODES = 10000
N_EDGES = 320000
D = 128
DH = 64


def _bdot(a, b):
    return jnp.dot(a.astype(jnp.bfloat16), b.astype(jnp.bfloat16),
                   preferred_element_type=jnp.float32)


def _layer_body(p_ref, w_ref, b_ref, out_ref):
    out_ref[...] = _bdot(p_ref[...], w_ref[...]) + b_ref[0:1, :]


def _final_body(p_ref, w2_ref, b2_ref, w3p_ref, b3_ref, out_ref):
    emb = _bdot(p_ref[...], w2_ref[...]) + b2_ref[0:1, :]
    out_ref[...] = _bdot(emb, w3p_ref[...]) + b3_ref[0:1, :]


def _make_agg_kernel(npad, e_pad, nc, ns, chunk):
    mesh = plsc.VectorSubcoreMesh(core_axis_name="c", subcore_axis_name="s")
    cpt = e_pad // ns
    nchunks = cpt // chunk
    npad2 = npad // nc
    nacc = npad2 + 128
    rpt = nacc // ns
    wpt = npad2 // ns

    @functools.partial(
        pl.kernel,
        out_type=jax.ShapeDtypeStruct((npad, D), jnp.float32),
        mesh=mesh,
        scratch_types=[
            pltpu.VMEM((chunk,), jnp.int32),
            pltpu.VMEM((chunk,), jnp.int32),
            pltpu.VMEM((chunk, D), jnp.float32),
            pltpu.VMEM_SHARED((nacc, D), jnp.float32),
            pltpu.SemaphoreType.DMA,
        ],
    )
    def agg_k(table, zeros, srcp, dstp, out, ia, ib, rows, agg_sh, sem):
        cid = lax.axis_index("c")
        sid = lax.axis_index("s")
        lo = cid * npad2
        pltpu.sync_copy(zeros.at[pl.ds(sid * rpt, rpt)],
                        agg_sh.at[pl.ds(sid * rpt, rpt)])
        plsc.subcore_barrier()

        base = sid * cpt

        def body(c, carry):
            off = base + c * chunk
            pltpu.sync_copy(srcp.at[pl.ds(off, chunk)], ia)
            pltpu.sync_copy(dstp.at[pl.ds(off, chunk)], ib)

            def remap(i, carry2):
                v = ib[pl.ds(i * 16, 16)] - lo
                keep = (v >= 0) & (v < npad2)
                ib[pl.ds(i * 16, 16)] = jnp.where(keep, v, npad2)
                return carry2

            lax.fori_loop(0, chunk // 16, remap, 0)
            pltpu.async_copy(table.at[ia], rows, sem).wait()
            pltpu.sync_copy(rows, agg_sh.at[ib], add=True)
            return carry

        lax.fori_loop(0, nchunks, body, 0)
        plsc.subcore_barrier()
        pltpu.sync_copy(agg_sh.at[pl.ds(sid * wpt, wpt)],
                        out.at[pl.ds(cid * npad2 + sid * wpt, wpt)])

    return agg_k


def _make_link_kernel(npad, e_pad, nc, ns):
    mesh = plsc.VectorSubcoreMesh(core_axis_name="c", subcore_axis_name="s")
    nw = nc * ns
    cpt = e_pad // nw

    @functools.partial(
        pl.kernel,
        out_type=jax.ShapeDtypeStruct((e_pad,), jnp.float32),
        mesh=mesh,
        scratch_types=[
            pltpu.VMEM((cpt,), jnp.int32),
            pltpu.VMEM((cpt,), jnp.int32),
            pltpu.VMEM((cpt,), jnp.float32),
            pltpu.VMEM((cpt,), jnp.float32),
            pltpu.VMEM((cpt,), jnp.float32),
            pltpu.VMEM_SHARED((npad,), jnp.float32),
            pltpu.VMEM_SHARED((npad,), jnp.float32),
            pltpu.SemaphoreType.DMA,
        ],
    )
    def link_k(ssv, stv, srcp, dstp, out, ia, ib, av, bv, ov,
               ss_sh, st_sh, sem):
        cid = lax.axis_index("c")
        sid = lax.axis_index("s")
        wid = sid * nc + cid
        base = wid * cpt

        @pl.when(sid == 0)
        def _():
            pltpu.sync_copy(ssv, ss_sh)
            pltpu.sync_copy(stv, st_sh)

        pltpu.sync_copy(srcp.at[pl.ds(base, cpt)], ia)
        pltpu.sync_copy(dstp.at[pl.ds(base, cpt)], ib)
        plsc.subcore_barrier()
        pltpu.async_copy(ss_sh.at[ia], av, sem).wait()
        pltpu.async_copy(st_sh.at[ib], bv, sem).wait()

        def vec(i, carry):
            a = av[pl.ds(i * 16, 16)]
            b = bv[pl.ds(i * 16, 16)]
            ov[pl.ds(i * 16, 16)] = 1.0 / (1.0 + jnp.exp(-(a + b)))
            return carry

        lax.fori_loop(0, cpt // 16, vec, 0)
        pltpu.sync_copy(ov, out.at[pl.ds(base, cpt)])

    return link_k


def kernel(node_features, edge_index, W1, b1, W2, b2, W3, b3):
    info = plsc.get_sparse_core_info()
    nc, ns = info.num_cores, info.num_subcores
    nw = nc * ns

    chunk = 640
    quant = ns * chunk
    e_pad = ((N_EDGES + quant - 1) // quant) * quant
    npad2 = ((N_NODES + nc - 1) // nc + 127) // 128 * 128
    npad = npad2 * nc

    src = edge_index[0].astype(jnp.int32)
    dst = edge_index[1].astype(jnp.int32)
    pad = jnp.full((e_pad - N_EDGES,), N_NODES, jnp.int32)
    src_p = jnp.concatenate([src, pad])
    dst_p = jnp.concatenate([dst, pad])

    xpad = jnp.zeros((npad, D), jnp.float32).at[:N_NODES].set(node_features)
    zeros = jnp.zeros((npad, D), jnp.float32)
    b1r = jnp.zeros((8, D), jnp.float32).at[0].set(b1)
    b2r = jnp.zeros((8, D), jnp.float32).at[0].set(b2)
    w3p = jnp.zeros((D, D), jnp.float32).at[:, 0].set(W3[:D, 0]).at[:, 1].set(W3[D:, 0])
    b3r = jnp.zeros((8, D), jnp.float32).at[0, 0].set(b3[0])

    agg_k = _make_agg_kernel(npad, e_pad, nc, ns, chunk)
    link_k = _make_link_kernel(npad, e_pad, nc, ns)

    layer = pl.pallas_call(
        _layer_body, out_shape=jax.ShapeDtypeStruct((npad, D), jnp.float32))
    final = pl.pallas_call(
        _final_body, out_shape=jax.ShapeDtypeStruct((npad, D), jnp.float32))

    p1 = agg_k(xpad, zeros, src_p, dst_p)
    hidden = layer(p1, W1, b1r)
    p2 = agg_k(hidden, zeros, src_p, dst_p)
    st = final(p2, W2, b2r, w3p, b3r)

    out = link_k(st[:, 0], st[:, 1], src_p, dst_p)
    return out[:N_EDGES].reshape(N_EDGES, 1)


# speedup vs baseline: 4.4398x; 1.5164x over previous
import functools

import jax
import jax.numpy as jnp
from jax import lax
from jax.experimental import pallas as pl
from jax.experimental.pallas import tpu as pltpu
from jax.experimental.pallas import tpu_sc as plsc

N_NODES = 10000
N_EDGES = 320000
D = 128
DH = 64


def _bdot(a, b):
    return jnp.dot(a.astype(jnp.bfloat16), b.astype(jnp.bfloat16),
                   preferred_element_type=jnp.float32)


def _layer_body(p_ref, w_ref, b_ref, out_ref):
    out_ref[...] = _bdot(p_ref[...], w_ref[...]) + b_ref[0:1, :]


def _final_body(p_ref, w2_ref, b2_ref, w3p_ref, b3_ref, out_ref):
    emb = _bdot(p_ref[...], w2_ref[...]) + b2_ref[0:1, :]
    out_ref[...] = _bdot(emb, w3p_ref[...]) + b3_ref[0:1, :]


def _make_agg_kernel(npad, e_pad, nc, ns, chunk):
    mesh = plsc.VectorSubcoreMesh(core_axis_name="c", subcore_axis_name="s")
    cpt = e_pad // ns
    nchunks = cpt // chunk
    npad2 = npad // nc
    nacc = npad2 + 128
    rpt = nacc // ns
    wpt = npad2 // ns

    @functools.partial(
        pl.kernel,
        out_type=jax.ShapeDtypeStruct((npad, D), jnp.float32),
        mesh=mesh,
        scratch_types=[
            pltpu.VMEM((cpt,), jnp.int32),
            pltpu.VMEM((cpt,), jnp.int32),
            pltpu.VMEM((chunk, D), jnp.float32),
            pltpu.VMEM((chunk, D), jnp.float32),
            pltpu.VMEM_SHARED((nacc, D), jnp.float32),
            pltpu.SemaphoreType.DMA,
            pltpu.SemaphoreType.DMA,
            pltpu.SemaphoreType.DMA,
        ],
    )
    def agg_k(table, zeros, srcp, dstp, out,
              ia, ib, r0, r1, agg_sh, sg0, sg1, ss):
        cid = lax.axis_index("c")
        sid = lax.axis_index("s")
        pltpu.sync_copy(zeros.at[pl.ds(sid * rpt, rpt)],
                        agg_sh.at[pl.ds(sid * rpt, rpt)])
        base = sid * cpt
        pltpu.sync_copy(srcp.at[pl.ds(base, cpt)], ia)
        pltpu.sync_copy(dstp.at[pl.ds(cid * e_pad + base, cpt)], ib)
        plsc.subcore_barrier()

        def body(c2, carry):
            a = 2 * c2 * chunk
            b = a + chunk
            ha = pltpu.async_copy(table.at[ia.at[pl.ds(a, chunk)]], r0, sg0)
            hb = pltpu.async_copy(table.at[ia.at[pl.ds(b, chunk)]], r1, sg1)
            ha.wait()
            hs = pltpu.async_copy(r0, agg_sh.at[ib.at[pl.ds(a, chunk)]], ss,
                                  add=True)
            hb.wait()
            hs.wait()
            pltpu.sync_copy(r1, agg_sh.at[ib.at[pl.ds(b, chunk)]], add=True)
            return carry

        lax.fori_loop(0, nchunks // 2, body, 0)
        plsc.subcore_barrier()
        pltpu.sync_copy(agg_sh.at[pl.ds(sid * wpt, wpt)],
                        out.at[pl.ds(cid * npad2 + sid * wpt, wpt)])

    return agg_k


def _make_link_kernel(npad, e_pad, nc, ns):
    mesh = plsc.VectorSubcoreMesh(core_axis_name="c", subcore_axis_name="s")
    nw = nc * ns
    cpt = e_pad // nw

    @functools.partial(
        pl.kernel,
        out_type=jax.ShapeDtypeStruct((e_pad,), jnp.float32),
        mesh=mesh,
        scratch_types=[
            pltpu.VMEM((cpt,), jnp.int32),
            pltpu.VMEM((cpt,), jnp.int32),
            pltpu.VMEM((cpt,), jnp.float32),
            pltpu.VMEM((cpt,), jnp.float32),
            pltpu.VMEM((cpt,), jnp.float32),
            pltpu.VMEM_SHARED((npad,), jnp.float32),
            pltpu.VMEM_SHARED((npad,), jnp.float32),
            pltpu.SemaphoreType.DMA,
        ],
    )
    def link_k(ssv, stv, srcp, dstp, out, ia, ib, av, bv, ov,
               ss_sh, st_sh, sem):
        cid = lax.axis_index("c")
        sid = lax.axis_index("s")
        wid = sid * nc + cid
        base = wid * cpt

        @pl.when(sid == 0)
        def _():
            pltpu.sync_copy(ssv, ss_sh)
            pltpu.sync_copy(stv, st_sh)

        pltpu.sync_copy(srcp.at[pl.ds(base, cpt)], ia)
        pltpu.sync_copy(dstp.at[pl.ds(base, cpt)], ib)
        plsc.subcore_barrier()
        pltpu.async_copy(ss_sh.at[ia], av, sem).wait()
        pltpu.async_copy(st_sh.at[ib], bv, sem).wait()

        def vec(i, carry):
            a = av[pl.ds(i * 16, 16)]
            b = bv[pl.ds(i * 16, 16)]
            ov[pl.ds(i * 16, 16)] = 1.0 / (1.0 + jnp.exp(-(a + b)))
            return carry

        lax.fori_loop(0, cpt // 16, vec, 0)
        pltpu.sync_copy(ov, out.at[pl.ds(base, cpt)])

    return link_k


def kernel(node_features, edge_index, W1, b1, W2, b2, W3, b3):
    info = plsc.get_sparse_core_info()
    nc, ns = info.num_cores, info.num_subcores
    nw = nc * ns

    chunk = 128
    quant = ns * chunk * 2
    e_pad = ((N_EDGES + quant - 1) // quant) * quant
    npad2 = ((N_NODES + nc - 1) // nc + 127) // 128 * 128
    npad = npad2 * nc

    src = edge_index[0].astype(jnp.int32)
    dst = edge_index[1].astype(jnp.int32)
    pad = jnp.full((e_pad - N_EDGES,), N_NODES, jnp.int32)
    src_p = jnp.concatenate([src, pad])
    dst_p = jnp.concatenate([dst, pad])
    locs = [jnp.where((dst_p >= c * npad2) & (dst_p < (c + 1) * npad2),
                      dst_p - c * npad2, npad2) for c in range(nc)]
    dst_loc = jnp.concatenate(locs)

    xpad = jnp.zeros((npad, D), jnp.float32).at[:N_NODES].set(node_features)
    zeros = jnp.zeros((npad, D), jnp.float32)
    b1r = jnp.zeros((8, D), jnp.float32).at[0].set(b1)
    b2r = jnp.zeros((8, D), jnp.float32).at[0].set(b2)
    w3p = jnp.zeros((D, D), jnp.float32).at[:, 0].set(W3[:D, 0]).at[:, 1].set(W3[D:, 0])
    b3r = jnp.zeros((8, D), jnp.float32).at[0, 0].set(b3[0])

    agg_k = _make_agg_kernel(npad, e_pad, nc, ns, chunk)
    link_k = _make_link_kernel(npad, e_pad, nc, ns)

    layer = pl.pallas_call(
        _layer_body, out_shape=jax.ShapeDtypeStruct((npad, D), jnp.float32))
    final = pl.pallas_call(
        _final_body, out_shape=jax.ShapeDtypeStruct((npad, D), jnp.float32))

    p1 = agg_k(xpad, zeros, src_p, dst_loc)
    hidden = layer(p1, W1, b1r)
    p2 = agg_k(hidden, zeros, src_p, dst_loc)
    st = final(p2, W2, b2r, w3p, b3r)

    out = link_k(st[:, 0], st[:, 1], src_p, dst_p)
    return out[:N_EDGES].reshape(N_EDGES, 1)
